# baseline (speedup 1.0000x reference)
import functools

import jax
import jax.numpy as jnp
from jax import lax
from jax.experimental import pallas as pl
from jax.experimental.pallas import tpu as pltpu
from jax.experimental.pallas import tpu_sc as plsc

_N = 16777216
_NSC = 7340032
_NTC = _N - _NSC
_NC = 2
_NS = 16
_NW = _NC * _NS
_PER_W = _NSC // _NW
_CHUNK = 16384
_NCHUNK = _PER_W // _CHUNK
_NBINS = 6
_L = 16
_UNROLL = 8
_EDGES = (-2.0, -1.0, 0.0, 1.0, 2.0)

_TC_COLS = 128
_TC_ROWS = 4096
_TC_BLK = _TC_ROWS * _TC_COLS
_TC_BLK0 = _NSC // _TC_BLK
_TC_STEPS = _NTC // _TC_BLK

_mesh = plsc.VectorSubcoreMesh(core_axis_name="c", subcore_axis_name="s")


@functools.partial(
    pl.kernel,
    mesh=_mesh,
    compiler_params=pltpu.CompilerParams(needs_layout_passes=False),
    out_type=jax.ShapeDtypeStruct((_NW * 256,), jnp.float32),
    scratch_types=[
        pltpu.VMEM((_CHUNK,), jnp.float32),
        pltpu.VMEM((_CHUNK,), jnp.float32),
        pltpu.VMEM((_CHUNK,), jnp.float32),
        pltpu.VMEM((_CHUNK,), jnp.float32),
        pltpu.VMEM((2 * _NBINS * _L,), jnp.float32),
        pltpu.VMEM((2 * _NBINS * _L,), jnp.float32),
        pltpu.SemaphoreType.DMA,
        pltpu.SemaphoreType.DMA,
    ],
)
def _sc_binned_partials(pred_hbm, targ_hbm, out_hbm,
                        pb0, tb0, pb1, tb1,
                        sacc, cacc,
                        sem0, sem1):
    wid = lax.axis_index("s") * _NC + lax.axis_index("c")
    zero16 = jnp.zeros((_L,), jnp.float32)
    for b in range(_NBINS):
        sacc[pl.ds(_L * b, _L)] = zero16
        cacc[pl.ds(_L * b, _L)] = zero16
    lane = lax.iota(jnp.int32, _L)
    lane6 = lane * _NBINS
    ones = jnp.ones((_L,), jnp.float32)
    base = wid * _PER_W

    def start(c, pb, tb, sem):
        off = base + c * _CHUNK
        pltpu.async_copy(pred_hbm.at[pl.ds(off, _CHUNK)], pb, sem)
        pltpu.async_copy(targ_hbm.at[pl.ds(off, _CHUNK)], tb, sem)

    def wait(pb, tb, sem):
        pltpu.make_async_copy(pred_hbm.at[pl.ds(0, _CHUNK)], pb, sem).wait()
        pltpu.make_async_copy(targ_hbm.at[pl.ds(0, _CHUNK)], tb, sem).wait()

    def process(pb, tb):
        @plsc.parallel_loop(0, _CHUNK // _L, 1, unroll=_UNROLL)
        def vbody(i):
            s = i * _L
            p = pb[pl.ds(s, _L)]
            t = tb[pl.ds(s, _L)]
            d = p - t
            u = jnp.abs(d)
            m = jnp.minimum(u, 1.0)
            elem = m * (u - 0.5 * m)
            tb3 = jnp.minimum(jnp.maximum(t + 3.0, 0.0), 5.5)
            bi = tb3.astype(jnp.int32)
            idx = bi * _L + lane
            plsc.addupdate_scatter(sacc, [idx], elem)
            plsc.addupdate_scatter(cacc, [idx], ones)

    start(0, pb0, tb0, sem0)

    def outer(i, carry):
        c0 = 2 * i
        start(c0 + 1, pb1, tb1, sem1)
        wait(pb0, tb0, sem0)
        process(pb0, tb0)

        @pl.when(c0 + 2 < _NCHUNK)
        def _():
            start(c0 + 2, pb0, tb0, sem0)

        wait(pb1, tb1, sem1)
        process(pb1, tb1)
        return carry

    lax.fori_loop(0, _NCHUNK // 2, outer, 0)

    obase = wid * 256
    pltpu.sync_copy(sacc.at[pl.ds(0, 96)], out_hbm.at[pl.ds(obase, 96)])
    pltpu.sync_copy(cacc.at[pl.ds(0, 96)], out_hbm.at[pl.ds(obase + 128, 96)])


def _tc_body(p_ref, t_ref, o_ref):
    @pl.when(pl.program_id(0) == 0)
    def _():
        for k in range(12):
            o_ref[k] = jnp.float32(0.0)

    zeros = jnp.zeros((8, 128), jnp.float32)
    ones = jnp.ones((8, 128), jnp.float32)

    def step(g, acc):
        ssum, scnt, stot = acc
        for j in range(4):
            r = (g * 4 + j) * 8
            p = p_ref[pl.ds(r, 8), :]
            t = t_ref[pl.ds(r, 8), :]
            d = p - t
            u = jnp.abs(d)
            m = jnp.minimum(u, 1.0)
            elem = m * (u - 0.5 * m)
            ssum = tuple(ssum[k] + jnp.where(t < e, elem, zeros)
                         for k, e in enumerate(_EDGES))
            scnt = tuple(scnt[k] + jnp.where(t < e, ones, zeros)
                         for k, e in enumerate(_EDGES))
            stot = stot + elem
        return (ssum, scnt, stot)

    init = ((zeros,) * 5, (zeros,) * 5, zeros)
    ssum, scnt, stot = lax.fori_loop(0, _TC_ROWS // 32, step, init)
    for k in range(5):
        o_ref[k] += jnp.sum(ssum[k])
        o_ref[6 + k] += jnp.sum(scnt[k])
    o_ref[5] += jnp.sum(stot)
    o_ref[11] += jnp.float32(_TC_BLK)


_tc_cumulative = pl.pallas_call(
    _tc_body,
    grid=(_TC_STEPS,),
    in_specs=[
        pl.BlockSpec((_TC_ROWS, _TC_COLS), lambda i: (_TC_BLK0 + i, 0)),
        pl.BlockSpec((_TC_ROWS, _TC_COLS), lambda i: (_TC_BLK0 + i, 0)),
    ],
    out_specs=pl.BlockSpec(memory_space=pltpu.SMEM),
    out_shape=jax.ShapeDtypeStruct((12,), jnp.float32),
)


def _combine_body(part_ref, cum_ref, o_ref):
    x = part_ref[...]
    total = jnp.float32(0.0)
    nbins = jnp.float32(0.0)
    prev_s = jnp.float32(0.0)
    prev_c = jnp.float32(0.0)
    for b in range(_NBINS):
        cur_s = cum_ref[b] if b < _NBINS - 1 else cum_ref[5]
        cur_c = cum_ref[6 + b] if b < _NBINS - 1 else cum_ref[11]
        sb = jnp.sum(x[:, 0, _L * b:_L * (b + 1)]) + (cur_s - prev_s)
        cb = jnp.sum(x[:, 1, _L * b:_L * (b + 1)]) + (cur_c - prev_c)
        prev_s, prev_c = cur_s, cur_c
        valid = cb > 0.0
        total = total + jnp.where(valid, sb / jnp.maximum(cb, 1.0), 0.0)
        nbins = nbins + valid.astype(jnp.float32)
    o_ref[0, 0] = total / nbins


def kernel(pred, target):
    parts = _sc_binned_partials(pred, target).reshape(_NW, 2, 128)
    p2 = pred.reshape(_N // _TC_COLS, _TC_COLS)
    t2 = target.reshape(_N // _TC_COLS, _TC_COLS)
    cums = _tc_cumulative(p2, t2)
    out = pl.pallas_call(
        _combine_body,
        in_specs=[
            pl.BlockSpec(memory_space=pltpu.VMEM),
            pl.BlockSpec(memory_space=pltpu.SMEM),
        ],
        out_shape=jax.ShapeDtypeStruct((1, 1), jnp.float32),
        out_specs=pl.BlockSpec(memory_space=pltpu.SMEM),
    )(parts, cums)
    return out[0, 0]


# speedup vs baseline: 1.0830x; 1.0830x over previous
import functools

import jax
import jax.numpy as jnp
from jax import lax
from jax.experimental import pallas as pl
from jax.experimental.pallas import tpu as pltpu
from jax.experimental.pallas import tpu_sc as plsc

_N = 16777216
_NSC = 8388608
_NTC = _N - _NSC
_NC = 2
_NS = 16
_NW = _NC * _NS
_PER_W = _NSC // _NW
_CHUNK = 16384
_NCHUNK = _PER_W // _CHUNK
_NBINS = 6
_L = 16
_UNROLL = 8
_EDGES = (-2.0, -1.0, 0.0, 1.0, 2.0)

_TC_COLS = 128
_TC_ROWS = 4096
_TC_BLK = _TC_ROWS * _TC_COLS
_TC_BLK0 = _NSC // _TC_BLK
_TC_STEPS = _NTC // _TC_BLK

_mesh = plsc.VectorSubcoreMesh(core_axis_name="c", subcore_axis_name="s")


@functools.partial(
    pl.kernel,
    mesh=_mesh,
    compiler_params=pltpu.CompilerParams(needs_layout_passes=False),
    out_type=jax.ShapeDtypeStruct((_NW * 256,), jnp.float32),
    scratch_types=[
        pltpu.VMEM((_CHUNK,), jnp.float32),
        pltpu.VMEM((_CHUNK,), jnp.float32),
        pltpu.VMEM((_CHUNK,), jnp.float32),
        pltpu.VMEM((_CHUNK,), jnp.float32),
        pltpu.VMEM((2 * _NBINS * _L,), jnp.float32),
        pltpu.VMEM((2 * _NBINS * _L,), jnp.float32),
        pltpu.SemaphoreType.DMA,
        pltpu.SemaphoreType.DMA,
    ],
)
def _sc_binned_partials(pred_hbm, targ_hbm, out_hbm,
                        pb0, tb0, pb1, tb1,
                        sacc, cacc,
                        sem0, sem1):
    wid = lax.axis_index("s") * _NC + lax.axis_index("c")
    zero16 = jnp.zeros((_L,), jnp.float32)
    for b in range(_NBINS):
        sacc[pl.ds(_L * b, _L)] = zero16
        cacc[pl.ds(_L * b, _L)] = zero16
    lane = lax.iota(jnp.int32, _L)
    lane6 = lane * _NBINS
    ones = jnp.ones((_L,), jnp.float32)
    base = wid * _PER_W

    def start(c, pb, tb, sem):
        off = base + c * _CHUNK
        pltpu.async_copy(pred_hbm.at[pl.ds(off, _CHUNK)], pb, sem)
        pltpu.async_copy(targ_hbm.at[pl.ds(off, _CHUNK)], tb, sem)

    def wait(pb, tb, sem):
        pltpu.make_async_copy(pred_hbm.at[pl.ds(0, _CHUNK)], pb, sem).wait()
        pltpu.make_async_copy(targ_hbm.at[pl.ds(0, _CHUNK)], tb, sem).wait()

    def process(pb, tb):
        @plsc.parallel_loop(0, _CHUNK // _L, 1, unroll=_UNROLL)
        def vbody(i):
            s = i * _L
            p = pb[pl.ds(s, _L)]
            t = tb[pl.ds(s, _L)]
            d = p - t
            u = jnp.abs(d)
            m = jnp.minimum(u, 1.0)
            elem = m * (u - 0.5 * m)
            tb3 = jnp.minimum(jnp.maximum(t + 3.0, 0.0), 5.5)
            bi = tb3.astype(jnp.int32)
            idx = bi * _L + lane
            plsc.addupdate_scatter(sacc, [idx], elem)
            plsc.addupdate_scatter(cacc, [idx], ones)

    start(0, pb0, tb0, sem0)

    def outer(i, carry):
        c0 = 2 * i
        start(c0 + 1, pb1, tb1, sem1)
        wait(pb0, tb0, sem0)
        process(pb0, tb0)

        @pl.when(c0 + 2 < _NCHUNK)
        def _():
            start(c0 + 2, pb0, tb0, sem0)

        wait(pb1, tb1, sem1)
        process(pb1, tb1)
        return carry

    lax.fori_loop(0, _NCHUNK // 2, outer, 0)

    obase = wid * 256
    pltpu.sync_copy(sacc.at[pl.ds(0, 96)], out_hbm.at[pl.ds(obase, 96)])
    pltpu.sync_copy(cacc.at[pl.ds(0, 96)], out_hbm.at[pl.ds(obase + 128, 96)])


def _tc_body(p_ref, t_ref, o_ref):
    @pl.when(pl.program_id(0) == 0)
    def _():
        for k in range(12):
            o_ref[k] = jnp.float32(0.0)

    zeros = jnp.zeros((8, 128), jnp.float32)
    ones = jnp.ones((8, 128), jnp.float32)

    def step(g, acc):
        ssum, scnt, stot = acc
        for j in range(4):
            r = (g * 4 + j) * 8
            p = p_ref[pl.ds(r, 8), :]
            t = t_ref[pl.ds(r, 8), :]
            d = p - t
            u = jnp.abs(d)
            m = jnp.minimum(u, 1.0)
            elem = m * (u - 0.5 * m)
            ssum = tuple(ssum[k] + jnp.where(t < e, elem, zeros)
                         for k, e in enumerate(_EDGES))
            scnt = tuple(scnt[k] + jnp.where(t < e, ones, zeros)
                         for k, e in enumerate(_EDGES))
            stot = stot + elem
        return (ssum, scnt, stot)

    init = ((zeros,) * 5, (zeros,) * 5, zeros)
    ssum, scnt, stot = lax.fori_loop(0, _TC_ROWS // 32, step, init)
    for k in range(5):
        o_ref[k] += jnp.sum(ssum[k])
        o_ref[6 + k] += jnp.sum(scnt[k])
    o_ref[5] += jnp.sum(stot)
    o_ref[11] += jnp.float32(_TC_BLK)


_tc_cumulative = pl.pallas_call(
    _tc_body,
    grid=(_TC_STEPS,),
    in_specs=[
        pl.BlockSpec((_TC_ROWS, _TC_COLS), lambda i: (_TC_BLK0 + i, 0)),
        pl.BlockSpec((_TC_ROWS, _TC_COLS), lambda i: (_TC_BLK0 + i, 0)),
    ],
    out_specs=pl.BlockSpec(memory_space=pltpu.SMEM),
    out_shape=jax.ShapeDtypeStruct((12,), jnp.float32),
)


def _combine_body(part_ref, cum_ref, o_ref):
    x = part_ref[...]
    total = jnp.float32(0.0)
    nbins = jnp.float32(0.0)
    prev_s = jnp.float32(0.0)
    prev_c = jnp.float32(0.0)
    for b in range(_NBINS):
        cur_s = cum_ref[b] if b < _NBINS - 1 else cum_ref[5]
        cur_c = cum_ref[6 + b] if b < _NBINS - 1 else cum_ref[11]
        sb = jnp.sum(x[:, 0, _L * b:_L * (b + 1)]) + (cur_s - prev_s)
        cb = jnp.sum(x[:, 1, _L * b:_L * (b + 1)]) + (cur_c - prev_c)
        prev_s, prev_c = cur_s, cur_c
        valid = cb > 0.0
        total = total + jnp.where(valid, sb / jnp.maximum(cb, 1.0), 0.0)
        nbins = nbins + valid.astype(jnp.float32)
    o_ref[0, 0] = total / nbins


def kernel(pred, target):
    parts = _sc_binned_partials(pred, target).reshape(_NW, 2, 128)
    p2 = pred.reshape(_N // _TC_COLS, _TC_COLS)
    t2 = target.reshape(_N // _TC_COLS, _TC_COLS)
    cums = _tc_cumulative(p2, t2)
    out = pl.pallas_call(
        _combine_body,
        in_specs=[
            pl.BlockSpec(memory_space=pltpu.VMEM),
            pl.BlockSpec(memory_space=pltpu.SMEM),
        ],
        out_shape=jax.ShapeDtypeStruct((1, 1), jnp.float32),
        out_specs=pl.BlockSpec(memory_space=pltpu.SMEM),
    )(parts, cums)
    return out[0, 0]
